# baseline (speedup 1.0000x reference)
import functools

import jax
import jax.numpy as jnp
import numpy as np
from jax import lax
from jax.experimental import pallas as pl
from jax.experimental.pallas import tpu as pltpu
from jax.experimental.pallas import tpu_sc as plsc

NC, NS = 2, 16
NW = NC * NS
BATCH = 2048
NPTS = 128
NSEG = NPTS - 1
HW = 28
NPIX = HW * HW
LANES = 16
NCHUNK = BATCH // LANES
NSPLIT = 2
NCHUNK_H = NCHUNK // NSPLIT
CPW = NCHUNK_H // NW
CWORDS = NPTS * LANES
BWORDS = NPIX * LANES
MSHIFT = 20

_EPS = np.float32(1e-7)
_PSET = np.float32(np.float32(1.0) - _EPS)
_A = np.float32(np.log(_EPS))
_B = np.float32(np.log(_PSET))
_A2 = np.float32(np.log(np.float32(np.float32(1.0) - _PSET)))

_MAGIC = np.zeros(32, np.int32)
for _d in range(1, HW):
    _MAGIC[_d] = (2**MSHIFT + _d - 1) // _d


def _sc_body(pxr, pyr, txr, tyr, bmr, magic_hbm, out_hbm,
             pxv0, pxv1, pyv0, pyv1, txv0, txv1, tyv0, tyv1,
             bmv0, bmv1, canvas, outv, magic_v,
             sem0, sem1):
    wid = lax.axis_index("c") * NS + lax.axis_index("s")

    lane = lax.iota(jnp.int32, LANES)
    zeros = jnp.zeros((LANES,), jnp.float32)
    ones = jnp.ones((LANES,), jnp.float32)

    pltpu.sync_copy(magic_hbm, magic_v)

    def zb(p, _):
        canvas[pl.ds(p * LANES, LANES)] = zeros
        return 0
    lax.fori_loop(0, NPIX, zb, 0)

    bufs = ((pxv0, pyv0, txv0, tyv0, bmv0, sem0),
            (pxv1, pyv1, txv1, tyv1, bmv1, sem1))

    def issue(j, buf):
        pxv, pyv, txv, tyv, bmv, sem = buf
        c = wid * CPW + j
        return (
            pltpu.async_copy(pxr.at[c], pxv, sem),
            pltpu.async_copy(pyr.at[c], pyv, sem),
            pltpu.async_copy(txr.at[c], txv, sem),
            pltpu.async_copy(tyr.at[c], tyv, sem),
            pltpu.async_copy(bmr.at[c], bmv, sem),
        )

    n_acc = zeros
    t_acc = zeros
    ta_acc = zeros
    mse_acc = zeros

    pending = issue(0, bufs[0])
    for j in range(CPW):
        pxv, pyv, txv, tyv, bmv, sem = bufs[j % 2]
        for h in pending:
            h.wait()
        if j + 1 < CPW:
            pending = issue(j + 1, bufs[(j + 1) % 2])

        def seg_body(k, _):
            o = k * LANES
            x0f = pxv[pl.ds(o, LANES)]
            y0f = pyv[pl.ds(o, LANES)]
            x1f = pxv[pl.ds(o + LANES, LANES)]
            y1f = pyv[pl.ds(o + LANES, LANES)]
            s = jnp.float32(HW - 1)
            x0 = (x0f * s).astype(jnp.int32)
            y0 = (y0f * s).astype(jnp.int32)
            x1 = (x1f * s).astype(jnp.int32)
            y1 = (y1f * s).astype(jnp.int32)

            steep = jnp.abs(y1 - y0) > jnp.abs(x1 - x0)
            ax0 = jnp.where(steep, y0, x0)
            ay0 = jnp.where(steep, x0, y0)
            ax1 = jnp.where(steep, y1, x1)
            ay1 = jnp.where(steep, x1, y1)
            swap = ax0 > ax1
            bx0 = jnp.where(swap, ax1, ax0)
            bx1 = jnp.where(swap, ax0, ax1)
            by0 = jnp.where(swap, ay1, ay0)
            by1 = jnp.where(swap, ay0, ay1)
            dx = bx1 - bx0
            dy = jnp.abs(by1 - by0)
            den = jnp.maximum(dx, 1)
            up = by0 < by1

            dyM = dy * plsc.load_gather(magic_v, [den])

            rr0 = jnp.where(steep, bx0, by0)
            cc0 = jnp.where(steep, by0, bx0)
            idx0 = (rr0 * HW + cc0) * LANES + lane
            step_x = jnp.where(steep, jnp.int32(HW * LANES), jnp.int32(LANES))
            sy_mag = jnp.where(steep, jnp.int32(LANES), jnp.int32(HW * LANES))
            step_y = jnp.where(up, sy_mag, -sy_mag)

            plsc.store_scatter(canvas, [idx0], ones)
            xacc = idx0
            for i in range(1, HW):
                xacc = xacc + step_x
                q = (dyM * i) >> MSHIFT
                m = dx >= i
                plsc.store_scatter(canvas, [xacc + q * step_y], ones, mask=m)
            return 0

        lax.fori_loop(0, NSEG, seg_body, 0)

        def red_body(p, accs):
            na, ta, taa = accs
            for u in range(4):
                q = p * (4 * LANES) + u * LANES
                cv = canvas[pl.ds(q, LANES)]
                canvas[pl.ds(q, LANES)] = zeros
                t = bmv[pl.ds(q, LANES)]
                na = na + cv
                ta = ta + cv * t
                taa = taa + t
            return (na, ta, taa)

        n_acc, t_acc, ta_acc = lax.fori_loop(
            0, NPIX // 4, red_body, (n_acc, t_acc, ta_acc))

        def mse_body(k, acc):
            for u in range(2):
                o = (k * 2 + u) * LANES
                d0 = pxv[pl.ds(o, LANES)] - txv[pl.ds(o, LANES)]
                d1 = pyv[pl.ds(o, LANES)] - tyv[pl.ds(o, LANES)]
                acc = acc + d0 * d0 + d1 * d1
            return acc

        mse_acc = lax.fori_loop(0, NPTS // 2, mse_body, mse_acc)

    outv[pl.ds(0, LANES)] = n_acc
    outv[pl.ds(LANES, LANES)] = t_acc
    outv[pl.ds(2 * LANES, LANES)] = ta_acc
    outv[pl.ds(3 * LANES, LANES)] = mse_acc
    pltpu.sync_copy(outv, out_hbm.at[wid])


@functools.partial(jax.jit, static_argnames=())
def kernel(pred_coords, target_coords, target_bitmap):
    def chunked(a):
        return a.reshape(NCHUNK_H, LANES, NPTS).transpose(0, 2, 1).reshape(
            NCHUNK_H, CWORDS)

    magic = jnp.asarray(_MAGIC)

    mesh = plsc.VectorSubcoreMesh(
        core_axis_name="c", subcore_axis_name="s",
        num_cores=NC, num_subcores=NS)

    run = pl.kernel(
        _sc_body,
        out_type=jax.ShapeDtypeStruct((NW, 4 * LANES), jnp.float32),
        mesh=mesh,
        compiler_params=pltpu.CompilerParams(needs_layout_passes=False),
        scratch_types=[
            pltpu.VMEM((CWORDS,), jnp.float32),
            pltpu.VMEM((CWORDS,), jnp.float32),
            pltpu.VMEM((CWORDS,), jnp.float32),
            pltpu.VMEM((CWORDS,), jnp.float32),
            pltpu.VMEM((CWORDS,), jnp.float32),
            pltpu.VMEM((CWORDS,), jnp.float32),
            pltpu.VMEM((CWORDS,), jnp.float32),
            pltpu.VMEM((CWORDS,), jnp.float32),
            pltpu.VMEM((BWORDS,), jnp.float32),
            pltpu.VMEM((BWORDS,), jnp.float32),
            pltpu.VMEM((BWORDS,), jnp.float32),
            pltpu.VMEM((4 * LANES,), jnp.float32),
            pltpu.VMEM((32,), jnp.int32),
            pltpu.SemaphoreType.DMA,
            pltpu.SemaphoreType.DMA,
        ],
    )

    halves = []
    hb = BATCH // NSPLIT
    for h in range(NSPLIT):
        sl = slice(h * hb, (h + 1) * hb)
        pxr = chunked(pred_coords[sl, :, 0])
        pyr = chunked(pred_coords[sl, :, 1])
        txr = chunked(target_coords[sl, :, 0])
        tyr = chunked(target_coords[sl, :, 1])
        bmr = target_bitmap[sl].reshape(
            NCHUNK_H, LANES, NPIX).transpose(0, 2, 1).reshape(
            NCHUNK_H, BWORDS)
        halves.append(run(pxr, pyr, txr, tyr, bmr, magic))
    parts = jnp.stack(halves).reshape(NSPLIT * NW, 4, LANES).sum(axis=(0, 2))
    n_set, t_set, t_all, sse = parts[0], parts[1], parts[2], parts[3]

    n_pix = np.float32(BATCH * NPIX)
    n_coord = np.float32(BATCH * NPTS * 2)
    coord_loss = sse / n_coord
    bce_sum = ((-_B) * n_pix - (_A - _B) * t_all
               + (_B - _A2) * n_set + (_A2 + _A - 2.0 * _B) * t_set)
    raster_loss = bce_sum / n_pix
    total_loss = (np.float32(1.0) * coord_loss
                  + np.float32(0.5) * raster_loss)
    return (coord_loss, raster_loss, total_loss)


# speedup vs baseline: 1.3177x; 1.3177x over previous
import functools

import jax
import jax.numpy as jnp
import numpy as np
from jax import lax
from jax.experimental import pallas as pl
from jax.experimental.pallas import tpu as pltpu
from jax.experimental.pallas import tpu_sc as plsc

NC, NS = 2, 16
NW = NC * NS
BATCH = 2048
NPTS = 128
NSEG = NPTS - 1
HW = 28
NPIX = HW * HW
LANES = 16
NCHUNK = BATCH // LANES
CPW = NCHUNK // NW
CWORDS = NPTS * LANES
BWORDS = NPIX * LANES
MSHIFT = 20

_EPS = np.float32(1e-7)
_PSET = np.float32(np.float32(1.0) - _EPS)
_A = np.float32(np.log(_EPS))
_B = np.float32(np.log(_PSET))
_A2 = np.float32(np.log(np.float32(np.float32(1.0) - _PSET)))

_MAGIC = np.zeros(32, np.int32)
for _d in range(1, HW):
    _MAGIC[_d] = (2**MSHIFT + _d - 1) // _d


def _sc_body(pxr, pyr, txr, tyr, bmr, magic_hbm, out_hbm,
             pxv0, pxv1, pyv0, pyv1, txv0, txv1, tyv0, tyv1,
             bmv0, bmv1, canvas, outv, magic_v,
             sem0, sem1):
    wid = lax.axis_index("c") * NS + lax.axis_index("s")

    lane = lax.iota(jnp.int32, LANES)
    zeros = jnp.zeros((LANES,), jnp.float32)
    ones = jnp.ones((LANES,), jnp.float32)

    pltpu.sync_copy(magic_hbm, magic_v)

    def zb(p, _):
        canvas[pl.ds(p * LANES, LANES)] = zeros
        return 0
    lax.fori_loop(0, NPIX, zb, 0)

    bufs = ((pxv0, pyv0, txv0, tyv0, bmv0, sem0),
            (pxv1, pyv1, txv1, tyv1, bmv1, sem1))

    def issue(j, buf):
        pxv, pyv, txv, tyv, bmv, sem = buf
        c = wid * CPW + j
        return (
            pltpu.async_copy(pxr.at[c], pxv, sem),
            pltpu.async_copy(pyr.at[c], pyv, sem),
            pltpu.async_copy(txr.at[c], txv, sem),
            pltpu.async_copy(tyr.at[c], tyv, sem),
            pltpu.async_copy(bmr.at[c], bmv, sem),
        )

    n_acc = zeros
    t_acc = zeros
    ta_acc = zeros
    mse_acc = zeros

    pending = issue(0, bufs[0])
    for j in range(CPW):
        pxv, pyv, txv, tyv, bmv, sem = bufs[j % 2]
        for h in pending:
            h.wait()
        if j + 1 < CPW:
            pending = issue(j + 1, bufs[(j + 1) % 2])

        def seg_body(k, _):
            o = k * LANES
            x0f = pxv[pl.ds(o, LANES)]
            y0f = pyv[pl.ds(o, LANES)]
            x1f = pxv[pl.ds(o + LANES, LANES)]
            y1f = pyv[pl.ds(o + LANES, LANES)]
            s = jnp.float32(HW - 1)
            x0 = (x0f * s).astype(jnp.int32)
            y0 = (y0f * s).astype(jnp.int32)
            x1 = (x1f * s).astype(jnp.int32)
            y1 = (y1f * s).astype(jnp.int32)

            steep = jnp.abs(y1 - y0) > jnp.abs(x1 - x0)
            ax0 = jnp.where(steep, y0, x0)
            ay0 = jnp.where(steep, x0, y0)
            ax1 = jnp.where(steep, y1, x1)
            ay1 = jnp.where(steep, x1, y1)
            swap = ax0 > ax1
            bx0 = jnp.where(swap, ax1, ax0)
            bx1 = jnp.where(swap, ax0, ax1)
            by0 = jnp.where(swap, ay1, ay0)
            by1 = jnp.where(swap, ay0, ay1)
            dx = bx1 - bx0
            dy = jnp.abs(by1 - by0)
            den = jnp.maximum(dx, 1)
            up = by0 < by1

            dyM = dy * plsc.load_gather(magic_v, [den])

            rr0 = jnp.where(steep, bx0, by0)
            cc0 = jnp.where(steep, by0, bx0)
            idx0 = (rr0 * HW + cc0) * LANES + lane
            step_x = jnp.where(steep, jnp.int32(HW * LANES), jnp.int32(LANES))
            sy_mag = jnp.where(steep, jnp.int32(LANES), jnp.int32(HW * LANES))
            step_y = jnp.where(up, sy_mag, -sy_mag)

            plsc.store_scatter(canvas, [idx0], ones)
            xacc = idx0
            for i in range(1, HW):
                xacc = xacc + step_x
                q = (dyM * i) >> MSHIFT
                m = dx >= i
                plsc.store_scatter(canvas, [xacc + q * step_y], ones, mask=m)
            return 0

        lax.fori_loop(0, NSEG, seg_body, 0)

        def red_body(p, accs):
            na, ta, taa = accs
            for u in range(4):
                q = p * (4 * LANES) + u * LANES
                cv = canvas[pl.ds(q, LANES)]
                canvas[pl.ds(q, LANES)] = zeros
                t = bmv[pl.ds(q, LANES)]
                na = na + cv
                ta = ta + cv * t
                taa = taa + t
            return (na, ta, taa)

        n_acc, t_acc, ta_acc = lax.fori_loop(
            0, NPIX // 4, red_body, (n_acc, t_acc, ta_acc))

        def mse_body(k, acc):
            for u in range(2):
                o = (k * 2 + u) * LANES
                d0 = pxv[pl.ds(o, LANES)] - txv[pl.ds(o, LANES)]
                d1 = pyv[pl.ds(o, LANES)] - tyv[pl.ds(o, LANES)]
                acc = acc + d0 * d0 + d1 * d1
            return acc

        mse_acc = lax.fori_loop(0, NPTS // 2, mse_body, mse_acc)

    outv[pl.ds(0, LANES)] = n_acc
    outv[pl.ds(LANES, LANES)] = t_acc
    outv[pl.ds(2 * LANES, LANES)] = ta_acc
    outv[pl.ds(3 * LANES, LANES)] = mse_acc
    pltpu.sync_copy(outv, out_hbm.at[wid])


@functools.partial(jax.jit, static_argnames=())
def kernel(pred_coords, target_coords, target_bitmap):
    def chunked(a):
        return a.reshape(NCHUNK, LANES, NPTS).transpose(0, 2, 1).reshape(
            NCHUNK, CWORDS)

    pxr = chunked(pred_coords[:, :, 0])
    pyr = chunked(pred_coords[:, :, 1])
    txr = chunked(target_coords[:, :, 0])
    tyr = chunked(target_coords[:, :, 1])
    bmr = target_bitmap.reshape(NCHUNK, LANES, NPIX).transpose(0, 2, 1).reshape(
        NCHUNK, BWORDS)
    magic = jnp.asarray(_MAGIC)

    mesh = plsc.VectorSubcoreMesh(
        core_axis_name="c", subcore_axis_name="s",
        num_cores=NC, num_subcores=NS)

    run = pl.kernel(
        _sc_body,
        out_type=jax.ShapeDtypeStruct((NW, 4 * LANES), jnp.float32),
        mesh=mesh,
        compiler_params=pltpu.CompilerParams(needs_layout_passes=False),
        scratch_types=[
            pltpu.VMEM((CWORDS,), jnp.float32),
            pltpu.VMEM((CWORDS,), jnp.float32),
            pltpu.VMEM((CWORDS,), jnp.float32),
            pltpu.VMEM((CWORDS,), jnp.float32),
            pltpu.VMEM((CWORDS,), jnp.float32),
            pltpu.VMEM((CWORDS,), jnp.float32),
            pltpu.VMEM((CWORDS,), jnp.float32),
            pltpu.VMEM((CWORDS,), jnp.float32),
            pltpu.VMEM((BWORDS,), jnp.float32),
            pltpu.VMEM((BWORDS,), jnp.float32),
            pltpu.VMEM((BWORDS,), jnp.float32),
            pltpu.VMEM((4 * LANES,), jnp.float32),
            pltpu.VMEM((32,), jnp.int32),
            pltpu.SemaphoreType.DMA,
            pltpu.SemaphoreType.DMA,
        ],
    )

    parts = run(pxr, pyr, txr, tyr, bmr, magic)
    parts = parts.reshape(NW, 4, LANES).sum(axis=(0, 2))
    n_set, t_set, t_all, sse = parts[0], parts[1], parts[2], parts[3]

    n_pix = np.float32(BATCH * NPIX)
    n_coord = np.float32(BATCH * NPTS * 2)
    coord_loss = sse / n_coord
    bce_sum = ((-_B) * n_pix - (_A - _B) * t_all
               + (_B - _A2) * n_set + (_A2 + _A - 2.0 * _B) * t_set)
    raster_loss = bce_sum / n_pix
    total_loss = (np.float32(1.0) * coord_loss
                  + np.float32(0.5) * raster_loss)
    return (coord_loss, raster_loss, total_loss)
